# baseline (speedup 1.0000x reference)
import functools

import jax
import jax.numpy as jnp
from jax import lax
from jax.experimental import pallas as pl
from jax.experimental.pallas import tpu as pltpu
from jax.experimental.pallas import tpu_sc as plsc

_BATCH = 4096
_HIST = 200
_HP = 208
_RW = 104
_D = 64
_NC = 2
_NS = 16
_NW = _NC * _NS
_EPW = _BATCH // _NW
_RPW = 2 * _EPW
_NBUF = 4


def _sc_pool_body(xr_hbm, table_hbm, out_hbm,
                  idx_v, buf0, buf1, buf2, buf3, row0_v, out_v,
                  sem0, sem1, sem2, sem3):
    bufs = (buf0, buf1, buf2, buf3)
    sems = (sem0, sem1, sem2, sem3)
    wid = lax.axis_index("s") * _NC + lax.axis_index("c")

    pltpu.sync_copy(xr_hbm.at[pl.ds(wid * _RPW, _RPW)], idx_v)
    pltpu.sync_copy(table_hbm.at[pl.ds(0, 8)], row0_v)

    for j in range(_NBUF):
        pltpu.async_copy(table_hbm.at[idx_v.at[j]], bufs[j], sems[j])

    lane = lax.iota(jnp.int32, 16)
    inv_n = jnp.float32(1.0 / _HIST)

    def elem(i, e):
        b_local = 2 * i + e
        acc = (jnp.zeros((16,), jnp.float32),) * 4
        cntv = jnp.zeros((16,), jnp.int32)
        for h in range(2):
            j = 2 * e + h
            r = 4 * i + j
            for k in range(6):
                c = idx_v[r, pl.ds(16 * k, 16)]
                cntv += plsc.all_reduce_population_count(c == 0)
            c = idx_v[r, pl.ds(88, 16)]
            cntv += plsc.all_reduce_population_count(
                (c == 0) & (lane >= 8))

            buf = bufs[j]
            pltpu.make_async_copy(
                table_hbm.at[pl.ds(0, _RW)], buf, sems[j]).wait()

            def row_add(jr, a, buf=buf):
                return (a[0] + buf[jr, pl.ds(0, 16)],
                        a[1] + buf[jr, pl.ds(16, 16)],
                        a[2] + buf[jr, pl.ds(32, 16)],
                        a[3] + buf[jr, pl.ds(48, 16)])

            acc = lax.fori_loop(0, _RW, row_add, acc)

            @pl.when(r + _NBUF < _RPW)
            def _(j=j, r=r):
                pltpu.async_copy(
                    table_hbm.at[idx_v.at[r + _NBUF]], bufs[j], sems[j])

        cnt = cntv.astype(jnp.float32)
        for k in range(4):
            val = (acc[k] - cnt * row0_v[0, pl.ds(16 * k, 16)]) * inv_n
            out_v[b_local, pl.ds(16 * k, 16)] = val

    def body(i, carry):
        elem(i, 0)
        elem(i, 1)
        return carry

    lax.fori_loop(0, _EPW // 2, body, 0)

    pltpu.sync_copy(out_v, out_hbm.at[pl.ds(wid * _EPW, _EPW)])


_sc_pool = functools.partial(
    pl.kernel,
    out_type=jax.ShapeDtypeStruct((_BATCH, _D), jnp.float32),
    mesh=plsc.VectorSubcoreMesh(core_axis_name="c", subcore_axis_name="s"),
    compiler_params=pltpu.CompilerParams(
        needs_layout_passes=False, use_tc_tiling_on_sc=False),
    scratch_types=[
        pltpu.VMEM((_RPW, _RW), jnp.int32),
        pltpu.VMEM((_RW, _D), jnp.float32),
        pltpu.VMEM((_RW, _D), jnp.float32),
        pltpu.VMEM((_RW, _D), jnp.float32),
        pltpu.VMEM((_RW, _D), jnp.float32),
        pltpu.VMEM((8, _D), jnp.float32),
        pltpu.VMEM((_EPW, _D), jnp.float32),
        pltpu.SemaphoreType.DMA,
        pltpu.SemaphoreType.DMA,
        pltpu.SemaphoreType.DMA,
        pltpu.SemaphoreType.DMA,
    ],
)(_sc_pool_body)


def _mlp_body(x_ref, w1_ref, b1_ref, w2_ref, b2_ref, o_ref):
    h = jnp.dot(x_ref[...], w1_ref[...], preferred_element_type=jnp.float32)
    h = jnp.maximum(h + b1_ref[...], 0.0)
    o_ref[...] = (
        jnp.dot(h, w2_ref[...], preferred_element_type=jnp.float32)
        + b2_ref[...])


def kernel(x, table, W1, b1, W2, b2):
    xi = x.astype(jnp.int32)
    xp = jnp.pad(xi, ((0, 0), (0, _HP - _HIST)))
    xr = xp.reshape(_BATCH * 2, _RW)

    pooled = _sc_pool(xr, table)

    ncls = W2.shape[1]
    w2p = jnp.pad(W2, ((0, 0), (0, _D - ncls)))
    b2p = jnp.pad(b2, (0, _D - ncls)).reshape(1, _D)
    out = pl.pallas_call(
        _mlp_body,
        out_shape=jax.ShapeDtypeStruct((_BATCH, _D), jnp.float32),
    )(pooled, W1, b1.reshape(1, -1), w2p, b2p)
    return out[:, :ncls]
